# baseline (speedup 1.0000x reference)
import math

import jax
import jax.numpy as jnp
from jax.experimental import pallas as pl

N_SWARM = 64
K_ACTIVE = 8
N_BASIS = 256
BLOCK_T = 4096

_SIN_Q = (25.132741095490235, -64.83582437360118, 67.07681466091034,
          -38.49910379702777, 14.069063708924725, -3.1999361834011273)


def _slsn_body(x_ref, f2_ref, p2_ref, amps_ref, biases_ref, gw_ref,
               fs_ref, fb_ref, out_ref):
    xb = x_ref[...]
    u = f2_ref[...] * xb + p2_ref[...]
    basis = jnp.sin(u)

    basis_sum = jnp.sum(basis, axis=0, keepdims=True)
    logits = jnp.dot(gw_ref[...], basis,
                     preferred_element_type=jnp.float32)

    m1 = jnp.max(logits, axis=0, keepdims=True)
    L = logits
    m = m1
    for _ in range(K_ACTIVE - 1):
        L = jnp.where(L == m, -jnp.inf, L)
        m = jnp.max(L, axis=0, keepdims=True)
    w = jnp.where(logits >= m, jnp.exp(logits - m1), 0.0)
    sum_w = jnp.sum(w, axis=0, keepdims=True)
    sum_wa = jnp.sum(w * amps_ref[...], axis=0, keepdims=True)
    sum_wb = jnp.sum(w * biases_ref[...], axis=0, keepdims=True)
    out = (basis_sum * sum_wa + sum_wb) / sum_w
    out_ref[...] = fs_ref[0, 0] * out + fb_ref[0, 0]


@jax.jit
def kernel(x, freqs, phases, amps, biases, gate_w, final_scale, final_bias):
    B = x.shape[0]
    grid = B // BLOCK_T
    xr = x.reshape(1, B)
    f2 = freqs.reshape(N_BASIS, 1)
    p2 = phases.reshape(N_BASIS, 1)
    amps_c = amps.reshape(N_SWARM, 1)
    biases_c = biases.reshape(N_SWARM, 1)
    fs = final_scale.reshape(1, 1)
    fb = final_bias.reshape(1, 1)

    out = pl.pallas_call(
        _slsn_body,
        grid=(grid,),
        in_specs=[
            pl.BlockSpec((1, BLOCK_T), lambda i: (0, i)),
            pl.BlockSpec((N_BASIS, 1), lambda i: (0, 0)),
            pl.BlockSpec((N_BASIS, 1), lambda i: (0, 0)),
            pl.BlockSpec((N_SWARM, 1), lambda i: (0, 0)),
            pl.BlockSpec((N_SWARM, 1), lambda i: (0, 0)),
            pl.BlockSpec((N_SWARM, N_BASIS), lambda i: (0, 0)),
            pl.BlockSpec((1, 1), lambda i: (0, 0)),
            pl.BlockSpec((1, 1), lambda i: (0, 0)),
        ],
        out_specs=pl.BlockSpec((1, BLOCK_T), lambda i: (0, i)),
        out_shape=jax.ShapeDtypeStruct((1, B), jnp.float32),
    )(xr, f2, p2, amps_c, biases_c, gate_w, fs, fb)
    return out.reshape(B, 1)
